# baseline (speedup 1.0000x reference)
import functools

import jax
import jax.numpy as jnp
from jax.experimental import pallas as pl

_NEG = -1e30


def _mm_nt(a, b):
    return jax.lax.dot_general(a, b, (((1,), (1,)), ((), ())),
                               preferred_element_type=jnp.float32)


def _layer_norm(xb, w, b):
    mu = jnp.mean(xb, axis=-1, keepdims=True)
    var = jnp.mean((xb - mu) ** 2, axis=-1, keepdims=True)
    return (xb - mu) / jnp.sqrt(var + 1e-5) * w + b


def _router_kernel(x_ref, ln1w_ref, ln1b_ref, attnW_ref, ln2w_ref, ln2b_ref,
                   gWp_ref, h_ref, y_ref, wpad_ref, *, num_experts):
    xb = x_ref[...]
    hb = xb + _mm_nt(_layer_norm(xb, ln1w_ref[...], ln1b_ref[...]),
                     attnW_ref[...])
    yb = _layer_norm(hb, ln2w_ref[...], ln2b_ref[...])
    logits = _mm_nt(yb, gWp_ref[...])
    tb = logits.shape[0]
    lane = jax.lax.broadcasted_iota(jnp.int32, (tb, 128), 1)
    lm = jnp.where(lane < num_experts, logits, _NEG)
    v1 = jnp.max(lm, axis=1, keepdims=True)
    i1 = jnp.min(jnp.where(lm == v1, lane, 127), axis=1, keepdims=True)
    lm2 = jnp.where(lane == i1, _NEG, lm)
    v2 = jnp.max(lm2, axis=1, keepdims=True)
    i2 = jnp.min(jnp.where(lm2 == v2, lane, 127), axis=1, keepdims=True)
    rw1 = jax.nn.sigmoid(v1 - v2)
    rw2 = 1.0 - rw1
    wpad = (jnp.where(lane == i1, rw1, 0.0)
            + jnp.where(lane == i2, rw2, 0.0))
    h_ref[...] = hb
    y_ref[...] = yb.astype(jnp.bfloat16)
    wpad_ref[...] = wpad


def _moe_kernel(y_ref, h_ref, wpad_ref, wg_ref, wu_ref, wd_ref, out_ref):
    e = pl.program_id(0)
    j = pl.program_id(1)
    onehot = (jax.lax.broadcasted_iota(jnp.int32, (128, 1), 0) == e
              ).astype(jnp.float32)
    wcol = jax.lax.dot_general(wpad_ref[...], onehot,
                               (((1,), (0,)), ((), ())),
                               preferred_element_type=jnp.float32)
    y = y_ref[...]
    g = _mm_nt(y, wg_ref[0].astype(jnp.bfloat16))
    u = _mm_nt(y, wu_ref[0].astype(jnp.bfloat16))
    a = (jax.nn.silu(g) * u).astype(jnp.bfloat16)
    contrib = _mm_nt(a, wd_ref[0].astype(jnp.bfloat16))
    val = contrib * wcol

    @pl.when(jnp.logical_and(e == 0, j == 0))
    def _init():
        out_ref[...] = h_ref[...] + val

    @pl.when(jnp.logical_or(e != 0, j != 0))
    def _acc():
        out_ref[...] += val


def kernel(x, ln1_w, ln1_b, attn_W, ln2_w, ln2_b, gate_W, Wg, Wu, Wd):
    T, H = x.shape
    E, I, _ = Wg.shape
    TB = min(256, T)
    JC = 256
    nj = I // JC

    gWp = jnp.pad(gate_W, ((0, 128 - E), (0, 0)))
    ln1w = ln1_w.reshape(1, H)
    ln1b = ln1_b.reshape(1, H)
    ln2w = ln2_w.reshape(1, H)
    ln2b = ln2_b.reshape(1, H)

    h, y, wpad = pl.pallas_call(
        functools.partial(_router_kernel, num_experts=E),
        grid=(T // TB,),
        in_specs=[
            pl.BlockSpec((TB, H), lambda i: (i, 0)),
            pl.BlockSpec((1, H), lambda i: (0, 0)),
            pl.BlockSpec((1, H), lambda i: (0, 0)),
            pl.BlockSpec((H, H), lambda i: (0, 0)),
            pl.BlockSpec((1, H), lambda i: (0, 0)),
            pl.BlockSpec((1, H), lambda i: (0, 0)),
            pl.BlockSpec((128, H), lambda i: (0, 0)),
        ],
        out_specs=[
            pl.BlockSpec((TB, H), lambda i: (i, 0)),
            pl.BlockSpec((TB, H), lambda i: (i, 0)),
            pl.BlockSpec((TB, 128), lambda i: (i, 0)),
        ],
        out_shape=[
            jax.ShapeDtypeStruct((T, H), jnp.float32),
            jax.ShapeDtypeStruct((T, H), jnp.bfloat16),
            jax.ShapeDtypeStruct((T, 128), jnp.float32),
        ],
    )(x, ln1w, ln1b, attn_W, ln2w, ln2b, gWp)

    out = pl.pallas_call(
        _moe_kernel,
        grid=(E, nj),
        in_specs=[
            pl.BlockSpec((T, H), lambda e, j: (0, 0)),
            pl.BlockSpec((T, H), lambda e, j: (0, 0)),
            pl.BlockSpec((T, 128), lambda e, j: (0, 0)),
            pl.BlockSpec((1, JC, H), lambda e, j: (e, j, 0)),
            pl.BlockSpec((1, JC, H), lambda e, j: (e, j, 0)),
            pl.BlockSpec((1, H, JC), lambda e, j: (e, 0, j)),
        ],
        out_specs=pl.BlockSpec((T, H), lambda e, j: (0, 0)),
        out_shape=jax.ShapeDtypeStruct((T, H), jnp.float32),
    )(y, h, wpad, Wg, Wu, Wd)
    return out


# speedup vs baseline: 1.1046x; 1.1046x over previous
import functools

import jax
import jax.numpy as jnp
from jax.experimental import pallas as pl
from jax.experimental.pallas import tpu as pltpu

_NEG = -1e30


def _mm_nt(a, b):
    return jax.lax.dot_general(a, b, (((1,), (1,)), ((), ())),
                               preferred_element_type=jnp.float32)


def _layer_norm(xb, w, b):
    mu = jnp.mean(xb, axis=-1, keepdims=True)
    var = jnp.mean((xb - mu) ** 2, axis=-1, keepdims=True)
    return (xb - mu) / jnp.sqrt(var + 1e-5) * w + b


def _router_kernel(x_ref, ln1w_ref, ln1b_ref, attnW_ref, ln2w_ref, ln2b_ref,
                   gWp_ref, h_ref, y_ref, wpad_ref, *, num_experts):
    xb = x_ref[...]
    hb = xb + _mm_nt(_layer_norm(xb, ln1w_ref[...], ln1b_ref[...]),
                     attnW_ref[...])
    yb = _layer_norm(hb, ln2w_ref[...], ln2b_ref[...])
    logits = _mm_nt(yb, gWp_ref[...])
    tb = logits.shape[0]
    lane = jax.lax.broadcasted_iota(jnp.int32, (tb, 128), 1)
    lm = jnp.where(lane < num_experts, logits, _NEG)
    v1 = jnp.max(lm, axis=1, keepdims=True)
    i1 = jnp.min(jnp.where(lm == v1, lane, 127), axis=1, keepdims=True)
    lm2 = jnp.where(lane == i1, _NEG, lm)
    v2 = jnp.max(lm2, axis=1, keepdims=True)
    i2 = jnp.min(jnp.where(lm2 == v2, lane, 127), axis=1, keepdims=True)
    rw1 = jax.nn.sigmoid(v1 - v2)
    rw2 = 1.0 - rw1
    wpad = (jnp.where(lane == i1, rw1, 0.0)
            + jnp.where(lane == i2, rw2, 0.0))
    h_ref[...] = hb
    y_ref[...] = yb.astype(jnp.bfloat16)
    wpad_ref[...] = wpad


def _moe_kernel(y_ref, h_ref, wpad_ref, wg_ref, wu_ref, wd_ref, out_ref,
                a_ref, *, nj, jc, hc):
    e = pl.program_id(0)
    s = pl.program_id(1)

    @pl.when(s < nj)
    def _up():
        onehot = (jax.lax.broadcasted_iota(jnp.int32, (128, 1), 0) == e
                  ).astype(jnp.float32)
        wcol = jax.lax.dot_general(wpad_ref[...], onehot,
                                   (((1,), (0,)), ((), ())),
                                   preferred_element_type=jnp.float32)
        y = y_ref[...]
        g = _mm_nt(y, wg_ref[0].astype(jnp.bfloat16))
        u = _mm_nt(y, wu_ref[0].astype(jnp.bfloat16))
        a = jax.nn.silu(g) * u * wcol
        a_ref[:, pl.ds(s * jc, jc)] = a.astype(jnp.bfloat16)

    @pl.when(s >= nj)
    def _down():
        hk = s - nj
        contrib = _mm_nt(a_ref[...], wd_ref[0].astype(jnp.bfloat16))

        @pl.when(e == 0)
        def _init():
            out_ref[:, pl.ds(hk * hc, hc)] = (
                h_ref[:, pl.ds(hk * hc, hc)] + contrib)

        @pl.when(e != 0)
        def _acc():
            out_ref[:, pl.ds(hk * hc, hc)] += contrib


def kernel(x, ln1_w, ln1_b, attn_W, ln2_w, ln2_b, gate_W, Wg, Wu, Wd):
    T, H = x.shape
    E, I, _ = Wg.shape
    TB = min(256, T)
    JC = 256
    nj = I // JC

    gWp = jnp.pad(gate_W, ((0, 128 - E), (0, 0)))
    ln1w = ln1_w.reshape(1, H)
    ln1b = ln1_b.reshape(1, H)
    ln2w = ln2_w.reshape(1, H)
    ln2b = ln2_b.reshape(1, H)

    h, y, wpad = pl.pallas_call(
        functools.partial(_router_kernel, num_experts=E),
        grid=(T // TB,),
        in_specs=[
            pl.BlockSpec((TB, H), lambda i: (i, 0)),
            pl.BlockSpec((1, H), lambda i: (0, 0)),
            pl.BlockSpec((1, H), lambda i: (0, 0)),
            pl.BlockSpec((H, H), lambda i: (0, 0)),
            pl.BlockSpec((1, H), lambda i: (0, 0)),
            pl.BlockSpec((1, H), lambda i: (0, 0)),
            pl.BlockSpec((128, H), lambda i: (0, 0)),
        ],
        out_specs=[
            pl.BlockSpec((TB, H), lambda i: (i, 0)),
            pl.BlockSpec((TB, H), lambda i: (i, 0)),
            pl.BlockSpec((TB, 128), lambda i: (i, 0)),
        ],
        out_shape=[
            jax.ShapeDtypeStruct((T, H), jnp.float32),
            jax.ShapeDtypeStruct((T, H), jnp.bfloat16),
            jax.ShapeDtypeStruct((T, 128), jnp.float32),
        ],
    )(x, ln1w, ln1b, attn_W, ln2w, ln2b, gWp)

    HC = 256
    nh = H // HC

    out = pl.pallas_call(
        functools.partial(_moe_kernel, nj=nj, jc=JC, hc=HC),
        grid=(E, nj + nh),
        in_specs=[
            pl.BlockSpec((T, H), lambda e, s: (0, 0)),
            pl.BlockSpec((T, H), lambda e, s: (0, 0)),
            pl.BlockSpec((T, 128), lambda e, s: (0, 0)),
            pl.BlockSpec((1, JC, H), lambda e, s: (e, jnp.minimum(s, nj - 1), 0)),
            pl.BlockSpec((1, JC, H), lambda e, s: (e, jnp.minimum(s, nj - 1), 0)),
            pl.BlockSpec((1, HC, I), lambda e, s: (e, jnp.maximum(s - nj, 0), 0)),
        ],
        out_specs=pl.BlockSpec((T, H), lambda e, s: (0, 0)),
        out_shape=jax.ShapeDtypeStruct((T, H), jnp.float32),
        scratch_shapes=[pltpu.VMEM((T, I), jnp.bfloat16)],
        compiler_params=pltpu.CompilerParams(
            vmem_limit_bytes=63 * 1024 * 1024),
    )(y, h, wpad, Wg, Wu, Wd)
    return out
